# baseline (speedup 1.0000x reference)
import dataclasses
import functools

import numpy as np

import jax
import jax.numpy as jnp
from jax import lax
from jax.experimental import pallas as pl
from jax.experimental.pallas import tpu as pltpu
from jax.experimental.pallas import tpu_sc as plsc

N = 10000
E = 160000
D = 256
HALFN = N // 2

NSUB = 16
SCH = 128
NSCH = 80
EPS = NSCH * SCH
EPAD = NSUB * EPS
GCH = 64
SCH2 = 32
CAP = 7296
MAXB = (CAP // (4 * GCH)) * 4
BASE_ROWS = 312

_PERM = np.empty((D,), np.int32)
_PERM[: D // 2] = 2 * np.arange(D // 2)
_PERM[D // 2:] = 2 * np.arange(D // 2) + 1

_mesh = plsc.VectorSubcoreMesh(core_axis_name="c", subcore_axis_name="s")

_sc_params = pltpu.CompilerParams()
if "needs_layout_passes" in pltpu.CompilerParams.__dataclass_fields__:
    _sc_params = dataclasses.replace(_sc_params, needs_layout_passes=False)
if "use_tc_tiling_on_sc" in pltpu.CompilerParams.__dataclass_fields__:
    _sc_params = dataclasses.replace(_sc_params, use_tc_tiling_on_sc=False)


@functools.partial(
    pl.kernel,
    out_type=jax.ShapeDtypeStruct((2, HALFN, D), jnp.float32),
    mesh=_mesh,
    scratch_types=[
        pltpu.VMEM((12, SCH), jnp.int32),
        pltpu.VMEM((CAP,), jnp.int32),
        pltpu.VMEM((CAP,), jnp.float32),
        pltpu.VMEM((4, GCH), jnp.int32),
        pltpu.VMEM((4, SCH2), jnp.int32),
        [pltpu.VMEM((GCH, D // 2), jnp.int32)] * 2,
        pltpu.VMEM((SCH2, D), jnp.float32),
        pltpu.VMEM((SCH2, D), jnp.float32),
        pltpu.VMEM_SHARED((HALFN, D), jnp.float32),
        [pltpu.SemaphoreType.DMA] * 4,
        [pltpu.SemaphoreType.DMA] * 2,
        pltpu.SemaphoreType.DMA,
        pltpu.SemaphoreType.DMA,
    ],
    compiler_params=_sc_params,
)
def _spmm_sc(edges_h, ebs_h, zeros_h, out_h,
             ering, fpacked, fvals, gidx, sidx, gbufs, s0, s1, acc,
             esems, gsems, ssem0, ssem1):
    c = lax.axis_index("c")
    s = lax.axis_index("s")

    r0 = s * BASE_ROWS
    pltpu.sync_copy(zeros_h.at[pl.ds(r0, BASE_ROWS)],
                    acc.at[pl.ds(r0, BASE_ROWS)])

    @pl.when(s == 0)
    def _():
        t0 = NSUB * BASE_ROWS
        pltpu.sync_copy(zeros_h.at[pl.ds(t0, 8)], acc.at[pl.ds(t0, 8)])

    zi = jnp.zeros((16,), jnp.int32)
    zf = jnp.zeros((16,), jnp.float32)

    @pl.loop(0, CAP, step=16)
    def _(i):
        fpacked[pl.ds(i, 16)] = zi
        fvals[pl.ds(i, 16)] = zf

    lo = c * HALFN

    def fetch_scan(q, slot):
        pltpu.async_copy(edges_h.at[s, q], ering.at[pl.ds(3 * slot, 3)],
                         esems[slot])

    def wait_scan(slot):
        pltpu.make_async_copy(edges_h.at[s, 0], ering.at[pl.ds(3 * slot, 3)],
                              esems[slot]).wait()

    for q in range(2):
        fetch_scan(q, q)

    def scan_chunk(slot, off):
        new_off = off
        for g in range(SCH // 16):
            sl = pl.ds(g * 16, 16)
            cols = ering[3 * slot, sl]
            rows = ering[3 * slot + 1, sl]
            vals = plsc.bitcast(ering[3 * slot + 2, sl], jnp.float32)
            lrow = rows - lo
            mask = (lrow >= 0) & (lrow < HALFN)
            pos = new_off + plsc.cumsum(mask.astype(jnp.int32)) - 1
            ok = mask & (pos < CAP)
            packed = cols | (lrow << 14)
            plsc.store_scatter(fpacked, [pos], packed, mask=ok)
            plsc.store_scatter(fvals, [pos], vals, mask=ok)
            new_off = new_off + plsc.all_reduce_population_count(mask)
        return new_off

    def scan_body(q, slot, off):
        wait_scan(slot)
        off = scan_chunk(slot, off)
        fetch_scan(q + 2, (slot + 2) % 4)
        return off

    off0 = jnp.zeros((16,), jnp.int32)
    for q in range(4):
        off0 = scan_body(q, q, off0)

    def _loop(q, off):
        for u in range(4):
            off = scan_body(q + u, u, off)
        return off

    off_vec = lax.fori_loop(1, NSCH // 4, lambda i, o: _loop(i * 4, o), off0)
    wait_scan((NSCH) % 4)
    wait_scan((NSCH + 1) % 4)

    cnt = jnp.minimum(jnp.max(off_vec), CAP)
    nblk = jnp.minimum((cnt + (4 * GCH - 1)) // (4 * GCH), MAXB // 4) * 4

    plsc.subcore_barrier()

    def build_gidx(t, k):
        o = t * GCH
        for h in range(GCH // 16):
            p0 = fpacked[pl.ds(o + h * 16, 16)]
            gidx[k, pl.ds(h * 16, 16)] = p0 & 16383

    def issue_gather(k, b):
        pltpu.async_copy(ebs_h.at[gidx.at[k]], gbufs[b], gsems[b])

    def wait_gather(b):
        pltpu.make_async_copy(ebs_h.at[gidx.at[0]], gbufs[b], gsems[b]).wait()

    def scale(gb, sb, roff, base):
        @pl.loop(0, SCH2, step=2)
        def _(e):
            for u in range(2):
                vv = plsc.load_gather(
                    fvals, [jnp.full((16,), base + e + u, jnp.int32)])
                src = gb.at[roff + e + u]
                dst = sb.at[e + u]
                for k in range(D // 32):
                    xi = src[pl.ds(k * 16, 16)]
                    xb = plsc.bitcast(xi, jnp.bfloat16)
                    a, b = plsc.unpack(
                        xb, format=plsc.PackFormat.INTERLEAVED,
                        preferred_element_type=jnp.float32)
                    dst[pl.ds(k * 16, 16)] = a * vv
                    dst[pl.ds(D // 2 + k * 16, 16)] = b * vv

    for t in range(2):
        build_gidx(t, t)
        issue_gather(t, t)

    def visit(t, u, first):
        b = u % 2
        wait_gather(b)
        for h in range(2):
            sb = s0 if h == 0 else s1
            ssem = ssem0 if h == 0 else ssem1
            base = t * GCH + h * SCH2
            ks = 2 * (u % 2) + h
            if not first:
                pltpu.make_async_copy(sb, acc.at[sidx.at[0]], ssem).wait()
            for g in range(SCH2 // 16):
                p0 = fpacked[pl.ds(base + g * 16, 16)]
                sidx[ks, pl.ds(g * 16, 16)] = lax.shift_right_logical(p0, 14)
            scale(gbufs[b], sb, h * SCH2, base)
            pltpu.async_copy(sb, acc.at[sidx.at[ks]], ssem, add=True)
        build_gidx(t + 2, (u + 2) % 4)
        issue_gather((u + 2) % 4, b)

    for u in range(4):
        visit(u, u, u < 1)

    @pl.loop(4, nblk, step=4)
    def _(tt):
        for u in range(4):
            visit(tt + u, u, False)

    for b in range(2):
        wait_gather(b)
    pltpu.make_async_copy(s0, acc.at[sidx.at[0]], ssem0).wait()
    pltpu.make_async_copy(s1, acc.at[sidx.at[0]], ssem1).wait()
    plsc.subcore_barrier()

    out_c = out_h.at[c]
    pltpu.sync_copy(acc.at[pl.ds(r0, BASE_ROWS)],
                    out_c.at[pl.ds(r0, BASE_ROWS)])

    @pl.when(s == 0)
    def _():
        t0 = NSUB * BASE_ROWS
        pltpu.sync_copy(acc.at[pl.ds(t0, 8)], out_c.at[pl.ds(t0, 8)])


def _tc_body(ls_ref, ebs_ref, ws_ref, wd_ref, o_ref):
    ls = ls_ref[...]
    eb = ebs_ref[...]
    li = ls + eb
    y = jnp.dot(li, ws_ref[...], preferred_element_type=jnp.float32)
    y += jnp.dot(ls * eb, wd_ref[...], preferred_element_type=jnp.float32)
    o_ref[...] = jnp.where(y >= 0, y, 0.2 * y)


_BM = 1000


def kernel(ebs, edge_index, edge_vals, W_side, W_dot):
    rows = edge_index[0]
    cols = edge_index[1]
    pad = EPAD - E
    rows2 = jnp.pad(rows, (0, pad), constant_values=-1)
    cols2 = jnp.pad(cols, (0, pad)).reshape(NSUB, NSCH, 1, SCH)
    vals2 = lax.bitcast_convert_type(
        jnp.pad(edge_vals, (0, pad)), jnp.int32).reshape(NSUB, NSCH, 1, SCH)
    rows2 = rows2.reshape(NSUB, NSCH, 1, SCH)
    edges = jnp.concatenate([cols2, rows2, vals2], axis=2)
    edges = jnp.pad(edges, ((0, 0), (0, 2), (0, 0), (0, 0)))
    ebs_i32 = lax.bitcast_convert_type(
        ebs.astype(jnp.bfloat16).reshape(N, D // 2, 2), jnp.int32)
    zeros = jnp.zeros((HALFN, D), jnp.float32)
    ebs_perm = ebs[:, _PERM]
    ws_perm = W_side[_PERM, :]
    wd_perm = W_dot[_PERM, :]

    ls_perm = _spmm_sc(edges, ebs_i32, zeros).reshape(N, D)

    out = pl.pallas_call(
        _tc_body,
        grid=(N // _BM,),
        in_specs=[
            pl.BlockSpec((_BM, D), lambda i: (i, 0)),
            pl.BlockSpec((_BM, D), lambda i: (i, 0)),
            pl.BlockSpec((D, D), lambda i: (0, 0)),
            pl.BlockSpec((D, D), lambda i: (0, 0)),
        ],
        out_specs=pl.BlockSpec((_BM, D), lambda i: (i, 0)),
        out_shape=jax.ShapeDtypeStruct((N, D), jnp.float32),
    )(ls_perm, ebs_perm, ws_perm, wd_perm)
    return out


# speedup vs baseline: 1.6097x; 1.6097x over previous
import dataclasses
import functools

import jax
import jax.numpy as jnp
from jax import lax
from jax.experimental import pallas as pl
from jax.experimental.pallas import tpu as pltpu
from jax.experimental.pallas import tpu_sc as plsc

N = 10000
E = 160000
D = 256
HALF = 128

NSUB = 16
CH = 64
NCH = 160
EPS = NCH * CH
EPAD = NSUB * EPS
NRING = 8
BASE_ROWS = 624

_mesh = plsc.VectorSubcoreMesh(core_axis_name="c", subcore_axis_name="s")

_sc_params = pltpu.CompilerParams()
if "needs_layout_passes" in pltpu.CompilerParams.__dataclass_fields__:
    _sc_params = dataclasses.replace(_sc_params, needs_layout_passes=False)


@functools.partial(
    pl.kernel,
    out_type=jax.ShapeDtypeStruct((2, N, HALF), jnp.float32),
    mesh=_mesh,
    scratch_types=[
        pltpu.VMEM((2 * NRING, CH), jnp.int32),
        pltpu.VMEM((EPS,), jnp.float32),
        pltpu.VMEM((CH, HALF), jnp.float32),
        pltpu.VMEM((CH, HALF), jnp.float32),
        pltpu.VMEM((CH, HALF), jnp.float32),
        pltpu.VMEM((CH, HALF), jnp.float32),
        pltpu.VMEM_SHARED((N, HALF), jnp.float32),
        [pltpu.SemaphoreType.DMA] * NRING,
        pltpu.SemaphoreType.DMA,
        pltpu.SemaphoreType.DMA,
        pltpu.SemaphoreType.DMA,
        pltpu.SemaphoreType.DMA,
    ],
    compiler_params=_sc_params,
)
def _spmm_sc(edges_h, vals_h, ebs_h, out_h,
             ering, vals_v, g0, g1, s0, s1, acc,
             esems, gsem0, gsem1, ssem0, ssem1):
    c = lax.axis_index("c")
    s = lax.axis_index("s")

    pltpu.sync_copy(vals_h.at[s], vals_v)

    r0 = s * BASE_ROWS
    ebs_c = ebs_h.at[c]
    pltpu.sync_copy(ebs_c.at[pl.ds(r0, BASE_ROWS)],
                    acc.at[pl.ds(r0, BASE_ROWS)])

    @pl.when(s < 2)
    def _():
        t0 = NSUB * BASE_ROWS + s * 8
        pltpu.sync_copy(ebs_c.at[pl.ds(t0, 8)], acc.at[pl.ds(t0, 8)])

    plsc.subcore_barrier()

    def fetch_edges(j, slot):
        pltpu.async_copy(edges_h.at[s, j], ering.at[pl.ds(2 * slot, 2)],
                         esems[slot])

    def wait_edges(slot):
        pltpu.make_async_copy(edges_h.at[s, 0], ering.at[pl.ds(2 * slot, 2)],
                              esems[slot]).wait()

    def issue_gather(slot, gb, gsem):
        pltpu.async_copy(ebs_c.at[ering.at[2 * slot]], gb, gsem)

    def scale(gb, sb, base):
        @pl.loop(0, CH, step=2)
        def _(e):
            for u in range(2):
                vv = plsc.load_gather(
                    vals_v, [jnp.full((16,), base + e + u, jnp.int32)])
                src = gb.at[e + u]
                dst = sb.at[e + u]
                for k in range(HALF // 16):
                    sl = pl.ds(k * 16, 16)
                    dst[sl] = src[sl] * vv

    for j in range(6):
        fetch_edges(j, j)
    wait_edges(0)
    issue_gather(0, g0, gsem0)
    wait_edges(1)
    issue_gather(1, g1, gsem1)

    def visit(j, u, first):
        gb, sb = (g0, s0) if u % 2 == 0 else (g1, s1)
        gsem = gsem0 if u % 2 == 0 else gsem1
        ssem = ssem0 if u % 2 == 0 else ssem1
        slot = u % NRING
        nxt = (u + 2) % NRING
        pltpu.make_async_copy(ebs_c.at[ering.at[2 * slot]], gb, gsem).wait()
        if not first:
            pltpu.make_async_copy(sb, acc.at[ering.at[1]], ssem).wait()
        fetch_edges(j + 6, (u + 6) % NRING)
        scale(gb, sb, j * CH)
        pltpu.async_copy(sb, acc.at[ering.at[2 * slot + 1]], ssem, add=True)
        wait_edges(nxt)
        issue_gather(nxt, gb, gsem)

    for u in range(NRING):
        visit(u, u, u < 2)

    @pl.loop(NRING, NCH, step=NRING)
    def _(jj):
        for u in range(NRING):
            visit(jj + u, u, False)

    pltpu.make_async_copy(ebs_c.at[ering.at[0]], g0, gsem0).wait()
    pltpu.make_async_copy(ebs_c.at[ering.at[2]], g1, gsem1).wait()
    pltpu.make_async_copy(s0, acc.at[ering.at[1]], ssem0).wait()
    pltpu.make_async_copy(s1, acc.at[ering.at[1]], ssem1).wait()
    wait_edges((NCH + 4) % NRING)
    wait_edges((NCH + 5) % NRING)
    plsc.subcore_barrier()

    out_c = out_h.at[c]
    pltpu.sync_copy(acc.at[pl.ds(r0, BASE_ROWS)],
                    out_c.at[pl.ds(r0, BASE_ROWS)])

    @pl.when(s < 2)
    def _():
        t0 = NSUB * BASE_ROWS + s * 8
        pltpu.sync_copy(acc.at[pl.ds(t0, 8)], out_c.at[pl.ds(t0, 8)])


def _tc_body(li0_ref, li1_ref, ebs_ref, ws_ref, wd_ref, o_ref):
    li = jnp.concatenate([li0_ref[0], li1_ref[0]], axis=1)
    eb = ebs_ref[...]
    ls = li - eb
    y = jnp.dot(li, ws_ref[...], preferred_element_type=jnp.float32)
    y += jnp.dot(ls * eb, wd_ref[...], preferred_element_type=jnp.float32)
    o_ref[...] = jnp.where(y >= 0, y, 0.2 * y)


_BM = 1000


def kernel(ebs, edge_index, edge_vals, W_side, W_dot):
    rows = edge_index[0]
    cols = edge_index[1]
    pad = EPAD - E
    rows2 = jnp.pad(rows, (0, pad)).reshape(NSUB, NCH, 1, CH)
    cols2 = jnp.pad(cols, (0, pad)).reshape(NSUB, NCH, 1, CH)
    edges = jnp.concatenate([cols2, rows2], axis=2)
    edges = jnp.pad(edges, ((0, 0), (0, 6), (0, 0), (0, 0)))
    vals2 = jnp.pad(edge_vals, (0, pad)).reshape(NSUB, EPS)
    ebs_halves = jnp.stack([ebs[:, :HALF], ebs[:, HALF:]])

    li_halves = _spmm_sc(edges, vals2, ebs_halves)

    out = pl.pallas_call(
        _tc_body,
        grid=(N // _BM,),
        in_specs=[
            pl.BlockSpec((1, _BM, HALF), lambda i: (0, i, 0)),
            pl.BlockSpec((1, _BM, HALF), lambda i: (1, i, 0)),
            pl.BlockSpec((_BM, D), lambda i: (i, 0)),
            pl.BlockSpec((D, D), lambda i: (0, 0)),
            pl.BlockSpec((D, D), lambda i: (0, 0)),
        ],
        out_specs=pl.BlockSpec((_BM, D), lambda i: (i, 0)),
        out_shape=jax.ShapeDtypeStruct((N, D), jnp.float32),
    )(li_halves, li_halves, ebs, W_side, W_dot)
    return out
